# baseline (speedup 1.0000x reference)
import functools

import jax
import jax.numpy as jnp
from jax import lax
from jax.experimental import pallas as pl
from jax.experimental.pallas import tpu as pltpu
from jax.experimental.pallas import tpu_sc as plsc

N_NODES = 4096
N_EDGES = 262144
M_CELLS = N_NODES * N_NODES
NW = 32
CPW = M_CELLS // NW
CHUNK = 65536
NCH = CPW // CHUNK
CAP = 16384
SENTINEL = 0x7FFFFFFF


def _mlp_body(ea_ref, r0_ref, r1_ref, w1_ref, b1_ref, w2_ref, b2_ref,
              dist_ref, lin_ref):
    ea = ea_ref[...]
    h = jnp.dot(ea, w1_ref[...].T, preferred_element_type=jnp.float32)
    h = h + b1_ref[...][None, :]
    h = 0.5 * h * (1.0 + jax.lax.erf(h * 0.7071067811865476))
    d = jnp.sum(h * w2_ref[...][0][None, :], axis=1) + b2_ref[...][0]
    dist_ref[...] = d[None, None, :]
    lin_ref[...] = r0_ref[...] * N_NODES + r1_ref[...]


def _mlp_pallas(edge_attr, r0, r1, w1_w, w1_b, w2_w, w2_b):
    E, K = edge_attr.shape
    BE = 8192
    nb = E // BE
    dist, lin = pl.pallas_call(
        _mlp_body,
        grid=(nb,),
        in_specs=[
            pl.BlockSpec((BE, K), lambda i: (i, 0)),
            pl.BlockSpec((1, 1, BE), lambda i: (i, 0, 0)),
            pl.BlockSpec((1, 1, BE), lambda i: (i, 0, 0)),
            pl.BlockSpec((K, K), lambda i: (0, 0)),
            pl.BlockSpec((K,), lambda i: (0,)),
            pl.BlockSpec((1, K), lambda i: (0, 0)),
            pl.BlockSpec((1,), lambda i: (0,)),
        ],
        out_specs=[
            pl.BlockSpec((1, 1, BE), lambda i: (i, 0, 0)),
            pl.BlockSpec((1, 1, BE), lambda i: (i, 0, 0)),
        ],
        out_shape=[
            jax.ShapeDtypeStruct((nb, 1, BE), jnp.float32),
            jax.ShapeDtypeStruct((nb, 1, BE), jnp.int32),
        ],
    )(edge_attr, r0, r1, w1_w, w1_b, w2_w, w2_b)
    return dist.reshape(E), lin.reshape(E)


def _sc_body(ukeys, svals, bnds, out, kbuf, vbuf, cellbuf, bnd_v):
    wid = lax.axis_index("s") * 2 + lax.axis_index("c")
    pltpu.sync_copy(bnds.at[pl.ds(wid * 8, 16)], bnd_v)
    bv = bnd_v[...]
    s_w = bv[0]
    start = jnp.minimum((s_w // 8) * 8, N_EDGES - CAP)
    pltpu.sync_copy(ukeys.at[pl.ds(start, CAP)], kbuf)
    pltpu.sync_copy(svals.at[pl.ds(start, CAP)], vbuf)
    zeros16 = jnp.zeros((16,), jnp.float32)

    def zero_body(i, carry):
        b0 = i * 256
        for u in range(16):
            cellbuf[pl.ds(b0 + u * 16, 16)] = zeros16
        return carry

    lax.fori_loop(0, CHUNK // 256, zero_body, 0)

    for c in range(NCH):
        base = wid * CPW + c * CHUNK
        g0 = jnp.maximum((bv[c] - start) // 16, 0)
        g1 = jnp.minimum((bv[c + 1] - start + 15) // 16, CAP // 16)

        def apply_body(j, carry):
            off = j * 16
            k = kbuf[pl.ds(off, 16)]
            v = vbuf[pl.ds(off, 16)]
            m = (k >= base) & (k < base + CHUNK)
            local = jnp.clip(k - base, 0, CHUNK - 1)
            plsc.store_scatter(cellbuf, [local], v, mask=m)
            return carry

        lax.fori_loop(g0, g1, apply_body, 0)
        pltpu.sync_copy(cellbuf, out.at[pl.ds(base, CHUNK)])

        def unscatter_body(j, carry):
            off = j * 16
            k = kbuf[pl.ds(off, 16)]
            m = (k >= base) & (k < base + CHUNK)
            local = jnp.clip(k - base, 0, CHUNK - 1)
            plsc.store_scatter(cellbuf, [local], zeros16, mask=m)
            return carry

        lax.fori_loop(g0, g1, unscatter_body, 0)


def _sc_apply(ukeys, svals, bnds):
    mesh = plsc.VectorSubcoreMesh(core_axis_name="c", subcore_axis_name="s")
    f = functools.partial(
        pl.kernel,
        out_type=jax.ShapeDtypeStruct((M_CELLS,), jnp.float32),
        mesh=mesh,
        compiler_params=pltpu.CompilerParams(needs_layout_passes=False),
        scratch_types=[
            pltpu.VMEM((CAP,), jnp.int32),
            pltpu.VMEM((CAP,), jnp.float32),
            pltpu.VMEM((CHUNK,), jnp.float32),
            pltpu.VMEM((16,), jnp.int32),
        ],
    )(_sc_body)
    return f(ukeys, svals, bnds)


def kernel(x, edge_idx, edge_attr, w1_w, w1_b, w2_w, w2_b):
    N = x.shape[0]
    E = edge_attr.shape[0]
    nb = E // 8192
    r0 = edge_idx[0].astype(jnp.int32).reshape(nb, 1, 8192)
    r1 = edge_idx[1].astype(jnp.int32).reshape(nb, 1, 8192)
    dist, lin = _mlp_pallas(edge_attr, r0, r1, w1_w, w1_b, w2_w, w2_b)
    skeys, svals = jax.lax.sort((lin, dist), num_keys=1, is_stable=False)
    skeys, svals = jax.lax.optimization_barrier((skeys, svals))
    nxt = jnp.concatenate([skeys[1:], jnp.full((1,), -1, jnp.int32)])
    ukeys = jnp.where(skeys != nxt, skeys, SENTINEL)
    tgt = jnp.arange(0, M_CELLS + 1, CHUNK, dtype=jnp.int32)
    lo = jnp.zeros((tgt.shape[0],), jnp.int32)
    hi = jnp.full((tgt.shape[0],), E, jnp.int32)
    for _ in range(19):
        mid = (lo + hi) // 2
        v = skeys[jnp.minimum(mid, E - 1)]
        go = (v < tgt) & (mid < E)
        lo = jnp.where(go, mid + 1, lo)
        hi = jnp.where(go, hi, mid)
    bnds = jnp.concatenate([lo, jnp.full((7,), E, jnp.int32)])
    out = _sc_apply(ukeys, svals, bnds)
    return out.reshape(N, N)


# speedup vs baseline: 1.1635x; 1.1635x over previous
import functools

import jax
import jax.numpy as jnp
from jax import lax
from jax.experimental import pallas as pl
from jax.experimental.pallas import tpu as pltpu
from jax.experimental.pallas import tpu_sc as plsc

N_NODES = 4096
N_EDGES = 262144
M_CELLS = N_NODES * N_NODES
NW = 32
CPW = M_CELLS // NW
CHUNK = 65536
NCH = CPW // CHUNK
CAP = 16384
SENTINEL = 0x7FFFFFFF


def _mlp_body(ea_ref, r0_ref, r1_ref, w1_ref, b1_ref, w2_ref, b2_ref,
              dist_ref, lin_ref):
    ea = ea_ref[...]
    h = jnp.dot(ea, w1_ref[...].T, preferred_element_type=jnp.float32)
    h = h + b1_ref[...][None, :]
    h = 0.5 * h * (1.0 + jax.lax.erf(h * 0.7071067811865476))
    d = jnp.sum(h * w2_ref[...][0][None, :], axis=1) + b2_ref[...][0]
    dist_ref[...] = d[None, None, :]
    lin_ref[...] = r0_ref[...] * N_NODES + r1_ref[...]


def _mlp_pallas(edge_attr, r0, r1, w1_w, w1_b, w2_w, w2_b):
    E, K = edge_attr.shape
    BE = 8192
    nb = E // BE
    dist, lin = pl.pallas_call(
        _mlp_body,
        grid=(nb,),
        in_specs=[
            pl.BlockSpec((BE, K), lambda i: (i, 0)),
            pl.BlockSpec((1, 1, BE), lambda i: (i, 0, 0)),
            pl.BlockSpec((1, 1, BE), lambda i: (i, 0, 0)),
            pl.BlockSpec((K, K), lambda i: (0, 0)),
            pl.BlockSpec((K,), lambda i: (0,)),
            pl.BlockSpec((1, K), lambda i: (0, 0)),
            pl.BlockSpec((1,), lambda i: (0,)),
        ],
        out_specs=[
            pl.BlockSpec((1, 1, BE), lambda i: (i, 0, 0)),
            pl.BlockSpec((1, 1, BE), lambda i: (i, 0, 0)),
        ],
        out_shape=[
            jax.ShapeDtypeStruct((nb, 1, BE), jnp.float32),
            jax.ShapeDtypeStruct((nb, 1, BE), jnp.int32),
        ],
    )(edge_attr, r0, r1, w1_w, w1_b, w2_w, w2_b)
    return dist.reshape(E), lin.reshape(E)


def _sc_body(ukeys, svals, bnds, out, kbuf, vbuf, cellbuf, bnd_v):
    wid = lax.axis_index("s") * 2 + lax.axis_index("c")
    pltpu.sync_copy(bnds.at[pl.ds(wid * 8, 16)], bnd_v)
    bv = bnd_v[...]
    s_w = bv[0]
    start = jnp.minimum((s_w // 8) * 8, N_EDGES - CAP)
    pltpu.sync_copy(ukeys.at[pl.ds(start, CAP)], kbuf)
    pltpu.sync_copy(svals.at[pl.ds(start, CAP)], vbuf)
    zeros16 = jnp.zeros((16,), jnp.float32)

    def zero_body(i, carry):
        b0 = i * 16
        for u in range(16):
            cellbuf[u, pl.ds(b0, 16)] = zeros16
        return carry

    lax.fori_loop(0, N_NODES // 16, zero_body, 0)

    rows_per_chunk = CHUNK // N_NODES
    for c in range(NCH):
        base = wid * CPW + c * CHUNK
        row0 = (wid * NCH + c) * rows_per_chunk
        g0 = jnp.maximum((bv[c] - start) // 16, 0)
        g1 = jnp.minimum((bv[c + 1] - start + 15) // 16, CAP // 16)

        def apply_body(j, carry):
            off = j * 16
            k = kbuf[pl.ds(off, 16)]
            v = vbuf[pl.ds(off, 16)]
            m = (k >= base) & (k < base + CHUNK)
            local = jnp.clip(k - base, 0, CHUNK - 1)
            plsc.store_scatter(
                cellbuf, [local >> 12, local & (N_NODES - 1)], v, mask=m)
            return carry

        lax.fori_loop(g0, g1, apply_body, 0)
        pltpu.sync_copy(cellbuf, out.at[pl.ds(row0, rows_per_chunk), :])

        def unscatter_body(j, carry):
            off = j * 16
            k = kbuf[pl.ds(off, 16)]
            m = (k >= base) & (k < base + CHUNK)
            local = jnp.clip(k - base, 0, CHUNK - 1)
            plsc.store_scatter(
                cellbuf, [local >> 12, local & (N_NODES - 1)], zeros16,
                mask=m)
            return carry

        lax.fori_loop(g0, g1, unscatter_body, 0)


def _sc_apply(ukeys, svals, bnds):
    mesh = plsc.VectorSubcoreMesh(core_axis_name="c", subcore_axis_name="s")
    f = functools.partial(
        pl.kernel,
        out_type=jax.ShapeDtypeStruct((N_NODES, N_NODES), jnp.float32),
        mesh=mesh,
        compiler_params=pltpu.CompilerParams(needs_layout_passes=False),
        scratch_types=[
            pltpu.VMEM((CAP,), jnp.int32),
            pltpu.VMEM((CAP,), jnp.float32),
            pltpu.VMEM((CHUNK // N_NODES, N_NODES), jnp.float32),
            pltpu.VMEM((16,), jnp.int32),
        ],
    )(_sc_body)
    return f(ukeys, svals, bnds)


def kernel(x, edge_idx, edge_attr, w1_w, w1_b, w2_w, w2_b):
    N = x.shape[0]
    E = edge_attr.shape[0]
    nb = E // 8192
    r0 = edge_idx[0].astype(jnp.int32).reshape(nb, 1, 8192)
    r1 = edge_idx[1].astype(jnp.int32).reshape(nb, 1, 8192)
    dist, lin = _mlp_pallas(edge_attr, r0, r1, w1_w, w1_b, w2_w, w2_b)
    skeys, svals = jax.lax.sort((lin, dist), num_keys=1, is_stable=False)
    skeys, svals = jax.lax.optimization_barrier((skeys, svals))
    nxt = jnp.concatenate([skeys[1:], jnp.full((1,), -1, jnp.int32)])
    ukeys = jnp.where(skeys != nxt, skeys, SENTINEL)
    edges = jnp.arange(0, M_CELLS + 1, CHUNK, dtype=jnp.int32)
    bounds = jnp.searchsorted(skeys, edges).astype(jnp.int32)
    bnds = jnp.concatenate([bounds, jnp.full((7,), E, jnp.int32)])
    return _sc_apply(ukeys, svals, bnds)


# speedup vs baseline: 1.4712x; 1.2645x over previous
import functools

import jax
import jax.numpy as jnp
from jax import lax
from jax.experimental import pallas as pl
from jax.experimental.pallas import tpu as pltpu
from jax.experimental.pallas import tpu_sc as plsc

N_NODES = 4096
N_EDGES = 262144
M_CELLS = N_NODES * N_NODES
NW = 32
CPW = M_CELLS // NW
CHUNK = 65536
NCH = CPW // CHUNK
CAP = 16384
SENTINEL = 0x7FFFFFFF


def _mlp_body(ea_ref, r0_ref, r1_ref, w1_ref, b1_ref, w2_ref, b2_ref,
              dist_ref, lin_ref):
    ea = ea_ref[...]
    z = jax.lax.dot_general(
        w1_ref[...], ea, (((1,), (1,)), ((), ())),
        preferred_element_type=jnp.float32)
    z = z + b1_ref[...][:, None]
    h = 0.5 * z * (1.0 + jax.lax.erf(z * 0.7071067811865476))
    d = jnp.sum(h * w2_ref[...][0][:, None], axis=0) + b2_ref[...][0]
    dist_ref[...] = d[None, None, :]
    lin_ref[...] = r0_ref[...] * N_NODES + r1_ref[...]


def _mlp_pallas(edge_attr, r0, r1, w1_w, w1_b, w2_w, w2_b):
    E, K = edge_attr.shape
    BE = 8192
    nb = E // BE
    dist, lin = pl.pallas_call(
        _mlp_body,
        grid=(nb,),
        in_specs=[
            pl.BlockSpec((BE, K), lambda i: (i, 0)),
            pl.BlockSpec((1, 1, BE), lambda i: (i, 0, 0)),
            pl.BlockSpec((1, 1, BE), lambda i: (i, 0, 0)),
            pl.BlockSpec((K, K), lambda i: (0, 0)),
            pl.BlockSpec((K,), lambda i: (0,)),
            pl.BlockSpec((1, K), lambda i: (0, 0)),
            pl.BlockSpec((1,), lambda i: (0,)),
        ],
        out_specs=[
            pl.BlockSpec((1, 1, BE), lambda i: (i, 0, 0)),
            pl.BlockSpec((1, 1, BE), lambda i: (i, 0, 0)),
        ],
        out_shape=[
            jax.ShapeDtypeStruct((nb, 1, BE), jnp.float32),
            jax.ShapeDtypeStruct((nb, 1, BE), jnp.int32),
        ],
    )(edge_attr, r0, r1, w1_w, w1_b, w2_w, w2_b)
    return dist.reshape(E), lin.reshape(E)


def _sc_body(ukeys, svals, bnds, out, kbuf, vbuf, cellbuf, bnd_v):
    wid = lax.axis_index("s") * 2 + lax.axis_index("c")
    pltpu.sync_copy(bnds.at[pl.ds(wid * 8, 16)], bnd_v)
    bv = bnd_v[...]
    s_w = bv[0]
    start = jnp.minimum((s_w // 8) * 8, N_EDGES - CAP)
    pltpu.sync_copy(ukeys.at[pl.ds(start, CAP)], kbuf)
    pltpu.sync_copy(svals.at[pl.ds(start, CAP)], vbuf)
    zeros16 = jnp.zeros((16,), jnp.float32)

    def zero_body(i, carry):
        b0 = i * 16
        for u in range(16):
            cellbuf[u, pl.ds(b0, 16)] = zeros16
        return carry

    lax.fori_loop(0, N_NODES // 16, zero_body, 0)

    rows_per_chunk = CHUNK // N_NODES
    for c in range(NCH):
        base = wid * CPW + c * CHUNK
        row0 = (wid * NCH + c) * rows_per_chunk
        g0 = jnp.maximum((bv[c] - start) // 16, 0)
        g1 = jnp.minimum((bv[c + 1] - start + 15) // 16, CAP // 16)

        def apply_body(j, carry):
            off = j * 16
            k = kbuf[pl.ds(off, 16)]
            v = vbuf[pl.ds(off, 16)]
            m = (k >= base) & (k < base + CHUNK)
            local = jnp.clip(k - base, 0, CHUNK - 1)
            plsc.store_scatter(
                cellbuf, [local >> 12, local & (N_NODES - 1)], v, mask=m)
            return carry

        lax.fori_loop(g0, g1, apply_body, 0)
        pltpu.sync_copy(cellbuf, out.at[pl.ds(row0, rows_per_chunk), :])

        def unscatter_body(j, carry):
            off = j * 16
            k = kbuf[pl.ds(off, 16)]
            m = (k >= base) & (k < base + CHUNK)
            local = jnp.clip(k - base, 0, CHUNK - 1)
            plsc.store_scatter(
                cellbuf, [local >> 12, local & (N_NODES - 1)], zeros16,
                mask=m)
            return carry

        lax.fori_loop(g0, g1, unscatter_body, 0)


def _sc_apply(ukeys, svals, bnds):
    mesh = plsc.VectorSubcoreMesh(core_axis_name="c", subcore_axis_name="s")
    f = functools.partial(
        pl.kernel,
        out_type=jax.ShapeDtypeStruct((N_NODES, N_NODES), jnp.float32),
        mesh=mesh,
        compiler_params=pltpu.CompilerParams(needs_layout_passes=False),
        scratch_types=[
            pltpu.VMEM((CAP,), jnp.int32),
            pltpu.VMEM((CAP,), jnp.float32),
            pltpu.VMEM((CHUNK // N_NODES, N_NODES), jnp.float32),
            pltpu.VMEM((16,), jnp.int32),
        ],
    )(_sc_body)
    return f(ukeys, svals, bnds)


def kernel(x, edge_idx, edge_attr, w1_w, w1_b, w2_w, w2_b):
    N = x.shape[0]
    E = edge_attr.shape[0]
    nb = E // 8192
    r0 = edge_idx[0].astype(jnp.int32).reshape(nb, 1, 8192)
    r1 = edge_idx[1].astype(jnp.int32).reshape(nb, 1, 8192)
    dist, lin = _mlp_pallas(edge_attr, r0, r1, w1_w, w1_b, w2_w, w2_b)
    skeys, svals = jax.lax.sort((lin, dist), num_keys=1, is_stable=False)
    skeys, svals = jax.lax.optimization_barrier((skeys, svals))
    nxt = jnp.concatenate([skeys[1:], jnp.full((1,), -1, jnp.int32)])
    ukeys = jnp.where(skeys != nxt, skeys, SENTINEL)
    edges = jnp.arange(0, M_CELLS + 1, CHUNK, dtype=jnp.int32)
    bounds = jnp.searchsorted(skeys, edges).astype(jnp.int32)
    bnds = jnp.concatenate([bounds, jnp.full((7,), E, jnp.int32)])
    return _sc_apply(ukeys, svals, bnds)
